# baseline (speedup 1.0000x reference)
import jax
import jax.numpy as jnp
from jax.experimental import pallas as pl


def _mm_body(u_ref, w_ref, o_ref):
    o_ref[...] = jnp.dot(u_ref[...], w_ref[...],
                         preferred_element_type=jnp.float32)


def _logits(U, W):
    M, K = U.shape
    K2, N = W.shape
    BM = 1024
    return pl.pallas_call(
        _mm_body,
        grid=(M // BM,),
        in_specs=[
            pl.BlockSpec((BM, K), lambda i: (i, 0)),
            pl.BlockSpec((K, N), lambda i: (0, 0)),
        ],
        out_specs=pl.BlockSpec((BM, N), lambda i: (i, 0)),
        out_shape=jax.ShapeDtypeStruct((M, N), jnp.float32),
    )(U, W)


def kernel(X, U, W):
    logits = _logits(U, W)
    M_T = jax.nn.softmax(logits, axis=0)
    A = M_T - jnp.min(M_T) + 1e-05
    kk = A.shape[1]
    _, idx_t = jax.lax.top_k(A.T, kk)
    indices = idx_t.T
    return X[:, indices]
